# baseline (speedup 1.0000x reference)
import jax
import jax.numpy as jnp
import numpy as np
from jax import lax
from jax.experimental import pallas as pl
from jax.experimental.pallas import tpu as pltpu
from jax.experimental.pallas import tpu_sc as plsc

INV_SQRT_2 = float(1.0 / np.sqrt(2.0))

N = 100000
E = 1600000
F = 16
NCORE = 2
NSUB = 16
NW = NCORE * NSUB

N_PAD = 100096
STRIPE = N_PAD // NSUB
WROWS = N_PAD * F // 128
WSTRIPE = WROWS // NSUB
ZROWS = 368
WCH = ZROWS // 8

JROWS = 16
SUPER = 25
ROWS_PER_TILE = JROWS * SUPER
E_PAD = NW * ROWS_PER_TILE * 128
EROWS = E_PAD // 128

def _sc_body(xall, src_r, dst_r, out, acc, idxs, idxd, rows0, rows1, rows2,
             rows3, zbuf, wbuf, gsem0, gsem1, gsem2, gsem3, ssem0, ssem1):
  cid = lax.axis_index("c")
  sid = lax.axis_index("s")
  wid = cid * NSUB + sid
  ebase = wid * ROWS_PER_TILE

  def _z(i, _):
    zbuf[i, :] = jnp.zeros((F,), jnp.float32)
    return 0
  lax.fori_loop(0, ZROWS, _z, 0)

  for p in range(3):
    xp = xall.at[p]
    for k in range(STRIPE // ZROWS):
      pltpu.sync_copy(zbuf, acc.at[pl.ds(sid * STRIPE + k * ZROWS, ZROWS)])
    plsc.subcore_barrier()

    bufs = (rows0, rows1, rows2, rows3)
    gsems = (gsem0, gsem1, gsem2, gsem3)
    ssems = (ssem0, ssem1)

    def _super(sb, _):
      rbase = ebase + sb * JROWS
      pltpu.sync_copy(src_r.at[pl.ds(rbase, JROWS)], idxs)
      pltpu.sync_copy(dst_r.at[pl.ds(rbase, JROWS)], idxd)

      def _gather(j):
        b = j % 4
        return pltpu.async_copy(xp.at[idxs.at[j]], bufs[b], gsems[b])

      def _scatter(j):
        return pltpu.async_copy(bufs[j % 4], acc.at[idxd.at[j]],
                                ssems[j % 2], add=True)

      gd = [None] * JROWS
      sd = [None] * JROWS
      for j in range(3):
        gd[j] = _gather(j)
      for j in range(JROWS):
        gd[j].wait()
        sd[j] = _scatter(j)
        if j + 3 < JROWS:
          if j - 1 >= 0:
            sd[j - 1].wait()
          gd[j + 3] = _gather(j + 3)
      for j in range(JROWS - 4, JROWS):
        sd[j].wait()
      return 0
    lax.fori_loop(0, SUPER, _super, 0)

    plsc.subcore_barrier()

    def _chunk(t, _):
      pltpu.sync_copy(acc.at[pl.ds(sid * STRIPE + t * ZROWS, ZROWS)], zbuf)

      def _wrow(r, _):
        for j in range(8):
          wbuf[r, pl.ds(j * F, F)] = zbuf[r * 8 + j, :]
        return 0
      lax.fori_loop(0, WCH, _wrow, 0)
      pltpu.sync_copy(wbuf,
                      out.at[cid, p, pl.ds(sid * WSTRIPE + t * WCH, WCH)])
      return 0
    lax.fori_loop(0, STRIPE // ZROWS, _chunk, 0)
    plsc.subcore_barrier()

    if p < 2:
      lax.fori_loop(0, ZROWS, _z, 0)


def _tc_body(x_ref, p_ref, nc_ref, mn_ref, me_ref, out_ref):
  x = x_ref[0]
  agg = p_ref[0, 0] + p_ref[1, 0]
  yn = jnp.dot(x, mn_ref[...], preferred_element_type=jnp.float32,
               precision=lax.Precision.HIGHEST)
  ya = jnp.dot(agg, me_ref[...], preferred_element_type=jnp.float32,
               precision=lax.Precision.HIGHEST)
  out_ref[0] = yn + nc_ref[...] * ya


def kernel(x, src, dst, norm_coeff, W_node, W_edge):
  xT = jnp.transpose(x[0], (2, 0, 1))

  pad = E_PAD - E
  src_r = jnp.concatenate(
      [src.astype(jnp.int32), jnp.zeros((pad,), jnp.int32)]).reshape(EROWS, 128)
  dst_r = jnp.concatenate(
      [dst.astype(jnp.int32), jnp.full((pad,), N, jnp.int32)]).reshape(EROWS, 128)

  mesh = plsc.VectorSubcoreMesh(core_axis_name="c", subcore_axis_name="s")
  partial = pl.kernel(
      _sc_body,
      out_type=jax.ShapeDtypeStruct((NCORE, 3, WROWS, 128), jnp.float32),
      mesh=mesh,
      compiler_params=pltpu.CompilerParams(use_tc_tiling_on_sc=False),
      scratch_types=[
          pltpu.VMEM_SHARED((N_PAD, F), jnp.float32),
          pltpu.VMEM((JROWS, 128), jnp.int32),
          pltpu.VMEM((JROWS, 128), jnp.int32),
          pltpu.VMEM((128, F), jnp.float32),
          pltpu.VMEM((128, F), jnp.float32),
          pltpu.VMEM((128, F), jnp.float32),
          pltpu.VMEM((128, F), jnp.float32),
          pltpu.VMEM((ZROWS, F), jnp.float32),
          pltpu.VMEM((WCH, 128), jnp.float32),
          pltpu.SemaphoreType.DMA,
          pltpu.SemaphoreType.DMA,
          pltpu.SemaphoreType.DMA,
          pltpu.SemaphoreType.DMA,
          pltpu.SemaphoreType.DMA,
          pltpu.SemaphoreType.DMA,
      ],
  )(xT, src_r, dst_r)

  eye8 = jnp.eye(8, dtype=jnp.float32)
  mn = jnp.kron(eye8, W_node.T) * INV_SQRT_2
  me = jnp.kron(eye8, W_edge.T) * INV_SQRT_2
  ncr = jnp.pad(jnp.repeat(norm_coeff, F),
                (0, (N_PAD - N) * F)).reshape(WROWS, 128)
  xw = jnp.pad(xT, ((0, 0), (0, N_PAD - N), (0, 0))).reshape(3, WROWS, 128)

  bn = 3128
  grid = (3, WROWS // bn)
  outw = pl.pallas_call(
      _tc_body,
      grid=grid,
      in_specs=[
          pl.BlockSpec((1, bn, 128), lambda i, b: (i, b, 0)),
          pl.BlockSpec((NCORE, 1, bn, 128), lambda i, b: (0, i, b, 0)),
          pl.BlockSpec((bn, 128), lambda i, b: (b, 0)),
          pl.BlockSpec((128, 128), lambda i, b: (0, 0)),
          pl.BlockSpec((128, 128), lambda i, b: (0, 0)),
      ],
      out_specs=pl.BlockSpec((1, bn, 128), lambda i, b: (i, b, 0)),
      out_shape=jax.ShapeDtypeStruct((3, WROWS, 128), jnp.float32),
  )(xw, partial, ncr, mn, me)

  out = outw.reshape(3, N_PAD, F)[:, :N]
  return jnp.transpose(out, (1, 2, 0))[None]


# speedup vs baseline: 1.3715x; 1.3715x over previous
import jax
import jax.numpy as jnp
import numpy as np
from jax import lax
from jax.experimental import pallas as pl
from jax.experimental.pallas import tpu as pltpu
from jax.experimental.pallas import tpu_sc as plsc

INV_SQRT_2 = float(1.0 / np.sqrt(2.0))

N = 100000
E = 1600000
F = 16
NCORE = 2
NSUB = 16
NW = NCORE * NSUB

N_PAD = 100096
STRIPE = N_PAD // NSUB
WROWS = N_PAD * F // 128
WSTRIPE = WROWS // NSUB
ZROWS = 368
WCH = ZROWS // 8

JROWS = 8
SUPER = 49
ROWS_PER_TILE = JROWS * SUPER
E_PAD = NW * ROWS_PER_TILE * 128
EROWS = E_PAD // 128

def _sc_body(xall, src_r, dst_r, out, acc, idxs, idxd, rows0, rows1, rows2,
             rows3, zbuf, wbuf, gsem0, gsem1, gsem2, gsem3, ssem0, ssem1):
  cid = lax.axis_index("c")
  sid = lax.axis_index("s")
  wid = cid * NSUB + sid
  ebase = wid * ROWS_PER_TILE

  def _z(i, _):
    zbuf[i, :] = jnp.zeros((F,), jnp.float32)
    return 0
  lax.fori_loop(0, ZROWS, _z, 0)

  for p in range(3):
    xp = xall.at[p]
    for k in range(STRIPE // ZROWS):
      pltpu.sync_copy(zbuf, acc.at[pl.ds(sid * STRIPE + k * ZROWS, ZROWS)])
    plsc.subcore_barrier()

    bufs = (rows0, rows1, rows2, rows3)
    gsems = (gsem0, gsem1, gsem2, gsem3)
    ssems = (ssem0, ssem1)

    def _super(sb, _):
      rbase = ebase + sb * JROWS
      pltpu.sync_copy(src_r.at[pl.ds(rbase, JROWS)], idxs)
      pltpu.sync_copy(dst_r.at[pl.ds(rbase, JROWS)], idxd)

      def _gather(j):
        b = j % 4
        return pltpu.async_copy(xp.at[idxs.at[j]], bufs[b], gsems[b])

      def _scatter(j):
        return pltpu.async_copy(bufs[j % 4], acc.at[idxd.at[j]],
                                ssems[j % 2], add=True)

      gd = [None] * JROWS
      sd = [None] * JROWS
      for j in range(3):
        gd[j] = _gather(j)
      for j in range(JROWS):
        gd[j].wait()
        sd[j] = _scatter(j)
        if j + 3 < JROWS:
          if j - 1 >= 0:
            sd[j - 1].wait()
          gd[j + 3] = _gather(j + 3)
      for j in range(JROWS - 4, JROWS):
        sd[j].wait()
      return 0
    lax.fori_loop(0, SUPER, _super, 0)

    plsc.subcore_barrier()

    def _chunk(t, _):
      pltpu.sync_copy(acc.at[pl.ds(sid * STRIPE + t * ZROWS, ZROWS)], zbuf)

      def _wrow(r, _):
        for j in range(8):
          wbuf[r, pl.ds(j * F, F)] = zbuf[r * 8 + j, :]
        return 0
      lax.fori_loop(0, WCH, _wrow, 0)
      pltpu.sync_copy(wbuf,
                      out.at[cid, p, pl.ds(sid * WSTRIPE + t * WCH, WCH)])
      return 0
    lax.fori_loop(0, STRIPE // ZROWS, _chunk, 0)
    plsc.subcore_barrier()

    if p < 2:
      lax.fori_loop(0, ZROWS, _z, 0)


def _tc_body(x_ref, p_ref, nc_ref, mn_ref, me_ref, out_ref):
  x = x_ref[0]
  agg = p_ref[0, 0] + p_ref[1, 0]
  yn = jnp.dot(x, mn_ref[...], preferred_element_type=jnp.float32,
               precision=lax.Precision.HIGHEST)
  ya = jnp.dot(agg, me_ref[...], preferred_element_type=jnp.float32,
               precision=lax.Precision.HIGHEST)
  out_ref[0] = yn + nc_ref[...] * ya


def kernel(x, src, dst, norm_coeff, W_node, W_edge):
  xT = jnp.transpose(x[0], (2, 0, 1))

  pad = E_PAD - E
  src_r = jnp.concatenate(
      [src.astype(jnp.int32), jnp.zeros((pad,), jnp.int32)]).reshape(EROWS, 128)
  dst_r = jnp.concatenate(
      [dst.astype(jnp.int32), jnp.full((pad,), N, jnp.int32)]).reshape(EROWS, 128)

  mesh = plsc.VectorSubcoreMesh(core_axis_name="c", subcore_axis_name="s")
  partial = pl.kernel(
      _sc_body,
      out_type=jax.ShapeDtypeStruct((NCORE, 3, WROWS, 128), jnp.float32),
      mesh=mesh,
      compiler_params=pltpu.CompilerParams(use_tc_tiling_on_sc=False),
      scratch_types=[
          pltpu.VMEM_SHARED((N_PAD, F), jnp.float32),
          pltpu.VMEM((JROWS, 128), jnp.int32),
          pltpu.VMEM((JROWS, 128), jnp.int32),
          pltpu.VMEM((128, F), jnp.float32),
          pltpu.VMEM((128, F), jnp.float32),
          pltpu.VMEM((128, F), jnp.float32),
          pltpu.VMEM((128, F), jnp.float32),
          pltpu.VMEM((ZROWS, F), jnp.float32),
          pltpu.VMEM((WCH, 128), jnp.float32),
          pltpu.SemaphoreType.DMA,
          pltpu.SemaphoreType.DMA,
          pltpu.SemaphoreType.DMA,
          pltpu.SemaphoreType.DMA,
          pltpu.SemaphoreType.DMA,
          pltpu.SemaphoreType.DMA,
      ],
  )(xT, src_r, dst_r)

  eye8 = jnp.eye(8, dtype=jnp.float32)
  mn = jnp.kron(eye8, W_node.T) * INV_SQRT_2
  me = jnp.kron(eye8, W_edge.T) * INV_SQRT_2
  ncr = jnp.pad(jnp.repeat(norm_coeff, F),
                (0, (N_PAD - N) * F)).reshape(WROWS, 128)
  xw = jnp.pad(xT, ((0, 0), (0, N_PAD - N), (0, 0))).reshape(3, WROWS, 128)

  bn = 3128
  grid = (3, WROWS // bn)
  outw = pl.pallas_call(
      _tc_body,
      grid=grid,
      in_specs=[
          pl.BlockSpec((1, bn, 128), lambda i, b: (i, b, 0)),
          pl.BlockSpec((NCORE, 1, bn, 128), lambda i, b: (0, i, b, 0)),
          pl.BlockSpec((bn, 128), lambda i, b: (b, 0)),
          pl.BlockSpec((128, 128), lambda i, b: (0, 0)),
          pl.BlockSpec((128, 128), lambda i, b: (0, 0)),
      ],
      out_specs=pl.BlockSpec((1, bn, 128), lambda i, b: (i, b, 0)),
      out_shape=jax.ShapeDtypeStruct((3, WROWS, 128), jnp.float32),
  )(xw, partial, ncr, mn, me)

  out = outw.reshape(3, N_PAD, F)[:, :N]
  return jnp.transpose(out, (1, 2, 0))[None]
